# Initial kernel scaffold; baseline (speedup 1.0000x reference)
import jax
import jax.numpy as jnp
from jax.experimental import pallas as pl


def kernel(z_t0_nodes, t_eval_points, edge_index, edge_weight, W0, b0, W1, b1, W2, b2):
    raise NotImplementedError("write your pallas kernel here")


# speedup vs baseline: 1.9914x; 1.9914x over previous
import functools

import jax
import jax.numpy as jnp
from jax import lax
from jax.experimental import pallas as pl
from jax.experimental.pallas import tpu as pltpu
from jax.experimental.pallas import tpu_sc as plsc

N = 10000
D = 128
E = 320000
NC = 2
NS = 16
NW = NC * NS
CH = 128
EPT = 10240
NCH = EPT // CH
NP = 10240
ROWS_PER_TILE = NP // NS

_segsum_kernel_kwargs = dict(
    out_type=jax.ShapeDtypeStruct((NC, NP, D), jnp.float32),
    scratch_types=[
        pltpu.VMEM((CH,), jnp.int32),
        pltpu.VMEM((CH,), jnp.int32),
        pltpu.VMEM((CH,), jnp.float32),
        pltpu.VMEM((CH, D), jnp.float32),
        pltpu.VMEM_SHARED((NP, D), jnp.float32),
        pltpu.SemaphoreType.DMA,
    ],
    compiler_params=pltpu.CompilerParams(needs_layout_passes=False),
)


def _segsum_body(h_hbm, src_hbm, dst_hbm, w_hbm, out_hbm,
            src_v, dst_v, w_v, rows_v, agg_sh, sem):
    c = lax.axis_index("c")
    s = lax.axis_index("s")
    wid = s * NC + c

    lanes = lax.broadcasted_iota(jnp.int32, (16,), 0)
    zeros16 = jnp.zeros((16,), jnp.float32)

    def _zfill(k, _):
        for g in range(D // 16):
            rows_v[k, pl.ds(g * 16, 16)] = zeros16
        return _
    lax.fori_loop(0, CH, _zfill, 0)

    row0 = s * ROWS_PER_TILE
    n_full = ROWS_PER_TILE // CH
    for j in range(n_full):
        pltpu.sync_copy(rows_v, agg_sh.at[pl.ds(row0 + j * CH, CH)])
    plsc.subcore_barrier()

    def _chunk(i, _):
        eb = wid * NCH + i
        pltpu.sync_copy(src_hbm.at[eb, 0], src_v)
        pltpu.sync_copy(dst_hbm.at[eb, 0], dst_v)
        pltpu.sync_copy(w_hbm.at[eb, 0], w_v)
        pltpu.async_copy(h_hbm.at[src_v], rows_v, sem).wait()

        def _scale(e, _):
            eidx = jnp.full((16,), e, jnp.int32)
            wv = plsc.load_gather(w_v, [eidx])
            for g in range(D // 16):
                sl = pl.ds(g * 16, 16)
                rows_v[e, sl] = rows_v[e, sl] * wv
            return _
        lax.fori_loop(0, CH, _scale, 0)

        pltpu.sync_copy(rows_v, agg_sh.at[dst_v], add=True)
        return _
    lax.fori_loop(0, NCH, _chunk, 0)
    plsc.subcore_barrier()

    for j in range(n_full):
        pltpu.sync_copy(agg_sh.at[pl.ds(row0 + j * CH, CH)], rows_v)
        pltpu.sync_copy(rows_v, out_hbm.at[c, pl.ds(row0 + j * CH, CH)])


_segsum_cache = []


def _segsum(*args):
    if not _segsum_cache:
        mesh = plsc.VectorSubcoreMesh(core_axis_name="c", subcore_axis_name="s",
                                      num_cores=NC, num_subcores=NS)
        _segsum_cache.append(functools.partial(
            pl.kernel, mesh=mesh, **_segsum_kernel_kwargs)(_segsum_body))
    return _segsum_cache[0](*args)


_RB = 2000


def _layer_mid_body(agg_ref, w_ref, b_ref, o_ref):
    x = agg_ref[0] + agg_ref[1]
    o_ref[...] = jnp.tanh(
        jnp.dot(x, w_ref[...], preferred_element_type=jnp.float32,
                precision=jax.lax.Precision.DEFAULT)
        + b_ref[...])


def _layer_last_body(agg_ref, w_ref, b_ref, y_ref, dt_ref, o_ref):
    x = agg_ref[0] + agg_ref[1]
    f = jnp.dot(x, w_ref[...], preferred_element_type=jnp.float32,
                precision=jax.lax.Precision.DEFAULT) + b_ref[...]
    o_ref[...] = y_ref[...] + dt_ref[0, 0] * f


_grid = (N // _RB,)
_agg_spec = pl.BlockSpec((NC, _RB, D), lambda i: (0, i, 0))
_w_spec = pl.BlockSpec((D, D), lambda i: (0, 0))
_b_spec = pl.BlockSpec((1, D), lambda i: (0, 0))
_row_spec = pl.BlockSpec((_RB, D), lambda i: (i, 0))
_dt_spec = pl.BlockSpec((1, 1), lambda i: (0, 0))
_out_sds = jax.ShapeDtypeStruct((N, D), jnp.float32)

_layer_mid = pl.pallas_call(
    _layer_mid_body, grid=_grid, out_shape=_out_sds,
    in_specs=[_agg_spec, _w_spec, _b_spec], out_specs=_row_spec)

_layer_last = pl.pallas_call(
    _layer_last_body, grid=_grid, out_shape=_out_sds,
    in_specs=[_agg_spec, _w_spec, _b_spec, _row_spec, _dt_spec],
    out_specs=_row_spec)


def _segsum_dbg(h, srcr, dstr, wr, edge_index, edge_weight):
    seg = jax.ops.segment_sum(h[edge_index[0]] * edge_weight[:, None],
                              edge_index[1], num_segments=N)
    out = jnp.zeros((NC, NP, D), jnp.float32)
    return out.at[0, :N].set(seg)


def kernel(z_t0_nodes, t_eval_points, edge_index, edge_weight,
           W0, b0, W1, b1, W2, b2):
    pad = EPT * NW - E
    src = jnp.concatenate([edge_index[0], jnp.zeros((pad,), jnp.int32)])
    dst = jnp.concatenate([edge_index[1], jnp.zeros((pad,), jnp.int32)])
    w = jnp.concatenate([edge_weight, jnp.zeros((pad,), jnp.float32)])
    srcr = src.reshape(NW * NCH, 1, CH)
    dstr = dst.reshape(NW * NCH, 1, CH)
    wr = w.reshape(NW * NCH, 1, CH)

    Ws = [W0, W1, W2]
    bs = [b0.reshape(1, D), b1.reshape(1, D), b2.reshape(1, D)]

    ys = [z_t0_nodes]
    y = z_t0_nodes
    T = t_eval_points.shape[0]
    n_sub = 4
    for i in range(T - 1):
        dt = ((t_eval_points[i + 1] - t_eval_points[i]) / n_sub).reshape(1, 1)
        for _ in range(n_sub):
            h = y
            for l in range(2):
                agg = _segsum(h, srcr, dstr, wr)
                h = _layer_mid(agg, Ws[l], bs[l])
            agg = _segsum(h, srcr, dstr, wr)
            y = _layer_last(agg, Ws[2], bs[2], y, dt)
        ys.append(y)
    return jnp.stack(ys, axis=0)


# speedup vs baseline: 3.0709x; 1.5420x over previous
import functools

import jax
import jax.numpy as jnp
from jax import lax
from jax.experimental import pallas as pl
from jax.experimental.pallas import tpu as pltpu
from jax.experimental.pallas import tpu_sc as plsc

N = 10000
D = 128
E = 320000
NC = 2
NS = 16
NW = NC * NS
CH = 64
EPT = 10240
NCH = EPT // CH
NP = 10000
ROWS_PER_TILE = NP // NS
_WB = [CH] * (ROWS_PER_TILE // CH) + ([ROWS_PER_TILE % CH] if ROWS_PER_TILE % CH else [])

_segsum_kernel_kwargs = dict(
    out_type=jax.ShapeDtypeStruct((NC, NP, D), jnp.float32),
    scratch_types=[
        pltpu.VMEM((NCH, 1, CH), jnp.int32),
        pltpu.VMEM((NCH, 1, CH), jnp.int32),
        pltpu.VMEM((EPT,), jnp.float32),
        pltpu.VMEM((2, CH, D), jnp.float32),
        pltpu.VMEM_SHARED((NP, D), jnp.float32),
        pltpu.SemaphoreType.DMA,
        pltpu.SemaphoreType.DMA,
    ],
    compiler_params=pltpu.CompilerParams(needs_layout_passes=False,
                                         use_tc_tiling_on_sc=False),
)


def _segsum_body(h_hbm, src_hbm, dst_hbm, w_hbm, out_hbm,
                 src_v, dst_v, w_v, rows_v, agg_sh, sem0, sem1):
    c = lax.axis_index("c")
    s = lax.axis_index("s")
    wid = s * NC + c
    sems = [sem0, sem1]

    zeros16 = jnp.zeros((16,), jnp.float32)

    pltpu.sync_copy(src_hbm.at[pl.ds(wid * NCH, NCH)], src_v)
    pltpu.sync_copy(dst_hbm.at[pl.ds(wid * NCH, NCH)], dst_v)
    pltpu.sync_copy(w_hbm.at[wid, 0], w_v)

    def _zfill(k, _):
        for g in range(D // 16):
            rows_v[0, k, pl.ds(g * 16, 16)] = zeros16
        return _
    lax.fori_loop(0, CH, _zfill, 0)

    row0 = s * ROWS_PER_TILE
    off = 0
    for sz in _WB:
        pltpu.sync_copy(rows_v.at[0, pl.ds(0, sz)],
                        agg_sh.at[pl.ds(row0 + off, sz)])
        off += sz
    plsc.subcore_barrier()

    pltpu.async_copy(h_hbm.at[src_v.at[0, 0]], rows_v.at[0], sems[0])

    @pl.loop(0, NCH, step=2)
    def _chunks(i):
        for b in range(2):
            j = i + b
            nxt = j + 1

            @pl.when(nxt < NCH)
            def _():
                pltpu.async_copy(h_hbm.at[src_v.at[nxt, 0]],
                                 rows_v.at[1 - b], sems[1 - b])

            pltpu.make_async_copy(h_hbm.at[src_v.at[j, 0]],
                                  rows_v.at[b], sems[b]).wait()

            def _scale(e):
                wv = plsc.load_gather(w_v, [jnp.full((16,), 0, jnp.int32)
                                            + j * CH + e])
                for g in range(D // 16):
                    sl = pl.ds(g * 16, 16)
                    rows_v[b, e, sl] = rows_v[b, e, sl] * wv
            plsc.parallel_loop(0, CH, 1, unroll=4)(_scale)

            pltpu.sync_copy(rows_v.at[b], agg_sh.at[dst_v.at[j, 0]], add=True)

    plsc.subcore_barrier()

    off = 0
    for sz in _WB:
        pltpu.sync_copy(agg_sh.at[pl.ds(row0 + off, sz)],
                        rows_v.at[0, pl.ds(0, sz)])
        pltpu.sync_copy(rows_v.at[0, pl.ds(0, sz)],
                        out_hbm.at[c, pl.ds(row0 + off, sz)])
        off += sz


_segsum_cache = []


def _segsum(*args):
    if not _segsum_cache:
        mesh = plsc.VectorSubcoreMesh(core_axis_name="c", subcore_axis_name="s",
                                      num_cores=NC, num_subcores=NS)
        _segsum_cache.append(functools.partial(
            pl.kernel, mesh=mesh, **_segsum_kernel_kwargs)(_segsum_body))
    return _segsum_cache[0](*args)


_RB = 2000


def _layer_mid_body(agg_ref, w_ref, b_ref, o_ref):
    x = agg_ref[0] + agg_ref[1]
    o_ref[...] = jnp.tanh(
        jnp.dot(x, w_ref[...], preferred_element_type=jnp.float32,
                precision=jax.lax.Precision.DEFAULT)
        + b_ref[...])


def _layer_last_body(agg_ref, w_ref, b_ref, y_ref, dt_ref, o_ref):
    x = agg_ref[0] + agg_ref[1]
    f = jnp.dot(x, w_ref[...], preferred_element_type=jnp.float32,
                precision=jax.lax.Precision.DEFAULT) + b_ref[...]
    o_ref[...] = y_ref[...] + dt_ref[0, 0] * f


_grid = (N // _RB,)
_agg_spec = pl.BlockSpec((NC, _RB, D), lambda i: (0, i, 0))
_w_spec = pl.BlockSpec((D, D), lambda i: (0, 0))
_b_spec = pl.BlockSpec((1, D), lambda i: (0, 0))
_row_spec = pl.BlockSpec((_RB, D), lambda i: (i, 0))
_dt_spec = pl.BlockSpec((1, 1), lambda i: (0, 0))
_out_sds = jax.ShapeDtypeStruct((N, D), jnp.float32)

_layer_mid = pl.pallas_call(
    _layer_mid_body, grid=_grid, out_shape=_out_sds,
    in_specs=[_agg_spec, _w_spec, _b_spec], out_specs=_row_spec)

_layer_last = pl.pallas_call(
    _layer_last_body, grid=_grid, out_shape=_out_sds,
    in_specs=[_agg_spec, _w_spec, _b_spec, _row_spec, _dt_spec],
    out_specs=_row_spec)


def _segsum_dbg(h, srcr, dstr, wr, edge_index, edge_weight):
    seg = jax.ops.segment_sum(h[edge_index[0]] * edge_weight[:, None],
                              edge_index[1], num_segments=N)
    out = jnp.zeros((NC, NP, D), jnp.float32)
    return out.at[0, :N].set(seg)


def kernel(z_t0_nodes, t_eval_points, edge_index, edge_weight,
           W0, b0, W1, b1, W2, b2):
    pad = EPT * NW - E
    src = jnp.concatenate([edge_index[0], jnp.zeros((pad,), jnp.int32)])
    dst = jnp.concatenate([edge_index[1], jnp.zeros((pad,), jnp.int32)])
    w = jnp.concatenate([edge_weight, jnp.zeros((pad,), jnp.float32)])
    srcr = src.reshape(NW * NCH, 1, CH)
    dstr = dst.reshape(NW * NCH, 1, CH)
    wr = w.reshape(NW, 1, EPT)

    Ws = [W0, W1, W2]
    bs = [b0.reshape(1, D), b1.reshape(1, D), b2.reshape(1, D)]

    ys = [z_t0_nodes]
    y = z_t0_nodes
    T = t_eval_points.shape[0]
    n_sub = 4
    for i in range(T - 1):
        dt = ((t_eval_points[i + 1] - t_eval_points[i]) / n_sub).reshape(1, 1)
        for _ in range(n_sub):
            h = y
            for l in range(2):
                agg = _segsum(h, srcr, dstr, wr)
                h = _layer_mid(agg, Ws[l], bs[l])
            agg = _segsum(h, srcr, dstr, wr)
            y = _layer_last(agg, Ws[2], bs[2], y, dt)
        ys.append(y)
    return jnp.stack(ys, axis=0)
